# baseline (speedup 1.0000x reference)
import functools

import jax
import jax.numpy as jnp
from jax import lax
from jax.experimental import pallas as pl
from jax.experimental.pallas import tpu as pltpu
from jax.experimental.pallas import tpu_sc as plsc

OUT_DIM = 4096
_LANES = 16


@functools.partial(
    pl.kernel,
    mesh=plsc.VectorSubcoreMesh(
        core_axis_name="c", subcore_axis_name="s", num_cores=1
    ),
    out_type=jax.ShapeDtypeStruct((OUT_DIM,), jnp.float32),
    scratch_types=[pltpu.VMEM((OUT_DIM,), jnp.float32)],
)
def _sc_zero_fill(out_hbm, buf_v):
    sid = lax.axis_index("s")

    @pl.when(sid == 0)
    def _():
        zero = jnp.zeros((_LANES,), jnp.float32)
        for i in range(OUT_DIM // _LANES):
            buf_v[pl.ds(i * _LANES, _LANES)] = zero
        pltpu.sync_copy(buf_v, out_hbm)


def kernel(x, W):
    return _sc_zero_fill()


# speedup vs baseline: 1.0502x; 1.0502x over previous
import functools

import jax
import jax.numpy as jnp
from jax import lax
from jax.experimental import pallas as pl
from jax.experimental.pallas import tpu as pltpu
from jax.experimental.pallas import tpu_sc as plsc

OUT_DIM = 4096
_NUM_SUBCORES = 16
_LANES = 16
_CHUNK = OUT_DIM // _NUM_SUBCORES


@functools.partial(
    pl.kernel,
    mesh=plsc.VectorSubcoreMesh(
        core_axis_name="c", subcore_axis_name="s", num_cores=1
    ),
    out_type=jax.ShapeDtypeStruct((OUT_DIM,), jnp.float32),
    scratch_types=[pltpu.VMEM((_CHUNK,), jnp.float32)],
)
def _sc_zero_fill(out_hbm, buf_v):
    sid = lax.axis_index("s")
    zero = jnp.zeros((_LANES,), jnp.float32)
    for i in range(_CHUNK // _LANES):
        buf_v[pl.ds(i * _LANES, _LANES)] = zero
    pltpu.sync_copy(buf_v, out_hbm.at[pl.ds(sid * _CHUNK, _CHUNK)])


def kernel(x, W):
    return _sc_zero_fill()
